# baseline (speedup 1.0000x reference)
import jax
import jax.numpy as jnp
from jax import lax
from jax.experimental import pallas as pl
from jax.experimental.pallas import tpu as pltpu
from jax.experimental.pallas import tpu_sc as plsc

_N = 4096
_K = 5
_LANES = 16
_NW = 32
_ROWS_PER_W = _N // _NW
_CHUNK = 8
_NCHUNK = _ROWS_PER_W // _CHUNK
_NCELL_VECS = _N // (_LANES * _LANES)
_TOPC = 6
_BLK = 256


def _denom_kernel(pred_ref, out_ref, flat_ref):
    i = pl.program_id(0)
    pred = pred_ref[...]
    b, n = pred.shape
    col = lax.broadcasted_iota(jnp.int32, (b, n), 1)
    row = lax.broadcasted_iota(jnp.int32, (b, n), 0) + i * b
    expp = jnp.where(col == row, 0.0, jnp.exp(pred))
    out_ref[...] = jnp.sum(expp, axis=1, keepdims=True)
    flat_ref[...] = pred.reshape(-1)


def _finale_kernel(vals_ref, denom_ref, out_ref):
    vals = vals_ref[...]
    denom = denom_ref[...]
    lane = lax.broadcasted_iota(jnp.int32, vals.shape, 1)
    term = -jnp.log(jnp.exp(vals) / denom + 1e-8)
    loss = jnp.sum(jnp.where(lane < _K, term, 0.0)) * (1.0 / _N)
    out_ref[...] = jnp.full((1, 1), loss, jnp.float32)


def _merge_desc(ak, av, bk, bv):
    brk = lax.rev(bk, (0,))
    brv = lax.rev(bv, (0,))
    m = ak >= brk
    hk = jnp.where(m, ak, brk)
    hv = jnp.where(m, av, brv)
    return plsc.sort_key_val(hk, hv, descending=True)


def _sc_top5(init_hbm, out_hbm, bufi, idx_v, si0, si1):
    c = lax.axis_index("c")
    s = lax.axis_index("s")
    wid = s * 2 + c
    row0 = wid * _ROWS_PER_W
    iota = lax.iota(jnp.int32, _LANES)
    neg = jnp.float32(-jnp.inf)

    def rows_of(ci):
        return pl.ds(row0 + ci * _CHUNK, _CHUNK)

    def do_row(bi, rl, ci):
        r = row0 + ci * _CHUNK + rl
        run = None
        for g in range(_NCELL_VECS):
            a = bi[rl, pl.ds(g * 256, _LANES)]
            b = bi[rl, pl.ds(g * 256 + 128, _LANES)]
            for t in range(1, 8):
                a = jnp.maximum(a, bi[rl, pl.ds(g * 256 + t * 16, _LANES)])
                b = jnp.maximum(b, bi[rl, pl.ds(g * 256 + 128 + t * 16,
                                                _LANES)])
            acc = jnp.maximum(a, b)
            if run is None:
                run = plsc.sort_key_val(acc, iota, descending=True)
            else:
                ids = iota + g * 16

                def _do_merge(run=run, acc=acc, ids=ids):
                    return _merge_desc(*run, *plsc.sort_key_val(
                        acc, ids, descending=True))

                def _keep(run=run):
                    return run

                run = lax.cond(jnp.max(acc) > run[0][_TOPC - 1],
                               _do_merge, _keep)
        top_keys, top_ids = run
        cand = None
        for j in range(_TOPC):
            cid = top_ids[j]
            g = lax.shift_right_logical(cid, 4)
            l = jnp.bitwise_and(cid, 15)
            pos = g * 256 + l + iota * 16

            def _scan_cell(pos=pos):
                v = plsc.load_gather(
                    bi, [jnp.full((_LANES,), rl, jnp.int32), pos])
                v = jnp.where(pos == r, neg, v)
                return plsc.sort_key_val(v, pos, descending=True)

            if cand is None:
                cand = _scan_cell()
            else:
                def _merge_cell(cand=cand, pos=pos):
                    return _merge_desc(*cand, *_scan_cell(pos))

                def _keep(cand=cand):
                    return cand

                cand = lax.cond(top_keys[j] > cand[0][_K - 1],
                                _merge_cell, _keep)
        _, cols = cand
        idxvec = r * _N + jnp.where(iota < _K, cols, r)
        idx_v[pl.ds((ci * _CHUNK + rl) * _LANES, _LANES)] = idxvec

    pltpu.async_copy(init_hbm.at[rows_of(0)], bufi.at[0], si0)
    pltpu.async_copy(init_hbm.at[rows_of(1)], bufi.at[1], si1)

    def half_body(h, carry):
        for slot, si in ((0, si0), (1, si1)):
            ci = h * 2 + slot
            pltpu.make_async_copy(init_hbm.at[rows_of(ci)], bufi.at[slot],
                                  si).wait()

            def row_body(rl, c2, _s=slot):
                do_row(bufi.at[_s], rl, c2)
                return c2

            lax.fori_loop(0, _CHUNK, row_body, ci)
            nci = ci + 2

            @pl.when(nci < _NCHUNK)
            def _(slot=slot, si=si, nci=nci):
                pltpu.async_copy(init_hbm.at[rows_of(nci)], bufi.at[slot],
                                 si)
        return carry

    lax.fori_loop(0, _NCHUNK // 2, half_body, 0)
    pltpu.sync_copy(idx_v,
                    out_hbm.at[pl.ds(row0 * _LANES,
                                     _ROWS_PER_W * _LANES)])


def _sc_gather(flat_hbm, idx_hbm, out_hbm, idx_v, vals1, vals2, sg):
    c = lax.axis_index("c")
    s = lax.axis_index("s")
    wid = s * 2 + c
    row0 = wid * _ROWS_PER_W
    pltpu.sync_copy(idx_hbm.at[pl.ds(row0 * _LANES, _ROWS_PER_W * _LANES)],
                    idx_v)
    handles = []
    for t in range(_ROWS_PER_W * _LANES // 128):
        handles.append(pltpu.async_copy(
            flat_hbm.at[idx_v.at[pl.ds(t * 128, 128)]],
            vals1.at[pl.ds(t * 128, 128)], sg))
    for hd in handles:
        hd.wait()

    def repack(rw, carry):
        vals2[rw, pl.ds(0, _LANES)] = vals1[pl.ds(rw * _LANES, _LANES)]
        return carry

    lax.fori_loop(0, _ROWS_PER_W, repack, 0)
    pltpu.sync_copy(vals2, out_hbm.at[pl.ds(row0, _ROWS_PER_W)])


def kernel(affinity_pred, affinity_init):
    denom, predflat = pl.pallas_call(
        _denom_kernel,
        grid=(_N // _BLK,),
        in_specs=[pl.BlockSpec((_BLK, _N), lambda i: (i, 0))],
        out_specs=[
            pl.BlockSpec((_BLK, 1), lambda i: (i, 0)),
            pl.BlockSpec((_BLK * _N,), lambda i: (i,)),
        ],
        out_shape=[
            jax.ShapeDtypeStruct((_N, 1), jnp.float32),
            jax.ShapeDtypeStruct((_N * _N,), jnp.float32),
        ],
    )(affinity_pred)

    mesh = plsc.VectorSubcoreMesh(core_axis_name="c", subcore_axis_name="s")
    idx = pl.kernel(
        _sc_top5,
        out_type=jax.ShapeDtypeStruct((_N * _LANES,), jnp.int32),
        mesh=mesh,
        compiler_params=pltpu.CompilerParams(needs_layout_passes=False),
        scratch_types=[
            pltpu.VMEM((2, _CHUNK, _N), jnp.float32),
            pltpu.VMEM((_ROWS_PER_W * _LANES,), jnp.int32),
            pltpu.SemaphoreType.DMA,
            pltpu.SemaphoreType.DMA,
        ],
    )(affinity_init)

    vals = pl.kernel(
        _sc_gather,
        out_type=jax.ShapeDtypeStruct((_N, _LANES), jnp.float32),
        mesh=mesh,
        compiler_params=pltpu.CompilerParams(needs_layout_passes=False),
        scratch_types=[
            pltpu.VMEM((_ROWS_PER_W * _LANES,), jnp.int32),
            pltpu.VMEM((_ROWS_PER_W * _LANES,), jnp.float32),
            pltpu.VMEM((_ROWS_PER_W, _LANES), jnp.float32),
            pltpu.SemaphoreType.DMA,
        ],
    )(predflat, idx)

    loss = pl.pallas_call(
        _finale_kernel,
        grid=(1,),
        in_specs=[
            pl.BlockSpec((_N, _LANES), lambda i: (0, 0)),
            pl.BlockSpec((_N, 1), lambda i: (0, 0)),
        ],
        out_specs=pl.BlockSpec((1, 1), lambda i: (0, 0)),
        out_shape=jax.ShapeDtypeStruct((1, 1), jnp.float32),
    )(vals, denom)
    return loss[0, 0]


# speedup vs baseline: 1.5556x; 1.5556x over previous
import jax
import jax.numpy as jnp
from jax import lax
from jax.experimental import pallas as pl
from jax.experimental.pallas import tpu as pltpu
from jax.experimental.pallas import tpu_sc as plsc

_N = 4096
_K = 5
_LANES = 16
_NW = 32
_ROWS_PER_W = _N // _NW
_CHUNK = 4
_NCHUNK = _ROWS_PER_W // _CHUNK
_NCELL_VECS = _N // (_LANES * _LANES)
_TOPC = 6


def _denom_kernel(pred_ref, out_ref):
    i = pl.program_id(0)
    pred = pred_ref[...]
    b, n = pred.shape
    col = lax.broadcasted_iota(jnp.int32, (b, n), 1)
    row = lax.broadcasted_iota(jnp.int32, (b, n), 0) + i * b
    expp = jnp.where(col == row, 0.0, jnp.exp(pred))
    out_ref[...] = jnp.sum(expp, axis=1, keepdims=True)


def _finale_kernel(vals_ref, denom_ref, out_ref):
    vals = vals_ref[...]
    denom = denom_ref[...]
    lane = lax.broadcasted_iota(jnp.int32, vals.shape, 1)
    term = -jnp.log(jnp.exp(vals) / denom + 1e-8)
    loss = jnp.sum(jnp.where(lane < _K, term, 0.0)) * (1.0 / _N)
    out_ref[...] = jnp.full((1, 1), loss, jnp.float32)


def _merge_desc(ak, av, bk, bv):
    brk = lax.rev(bk, (0,))
    brv = lax.rev(bv, (0,))
    m = ak >= brk
    hk = jnp.where(m, ak, brk)
    hv = jnp.where(m, av, brv)
    return plsc.sort_key_val(hk, hv, descending=True)


def _merge_tree(pairs):
    while len(pairs) > 1:
        nxt = []
        for j in range(0, len(pairs) - 1, 2):
            nxt.append(_merge_desc(*pairs[j], *pairs[j + 1]))
        if len(pairs) % 2:
            nxt.append(pairs[-1])
        pairs = nxt
    return pairs[0]


def _sc_top5_gather(init_hbm, pred_hbm, out_hbm, bufi, bufp, vals_v,
                    si0, si1, sp0, sp1):
    c = lax.axis_index("c")
    s = lax.axis_index("s")
    wid = s * 2 + c
    row0 = wid * _ROWS_PER_W
    iota = lax.iota(jnp.int32, _LANES)
    neg = jnp.float32(-jnp.inf)

    def rows_of(ci):
        return pl.ds(row0 + ci * _CHUNK, _CHUNK)

    def do_row(bi, bp, rl, ci):
        r = row0 + ci * _CHUNK + rl
        run = None
        for g0 in range(0, _NCELL_VECS, 2):
            local = []
            for g in (g0, g0 + 1):
                a = bi[rl, pl.ds(g * 256, _LANES)]
                b = bi[rl, pl.ds(g * 256 + 128, _LANES)]
                for t in range(1, 8):
                    a = jnp.maximum(a, bi[rl, pl.ds(g * 256 + t * 16,
                                                    _LANES)])
                    b = jnp.maximum(b, bi[rl, pl.ds(g * 256 + 128 + t * 16,
                                                    _LANES)])
                acc = jnp.maximum(a, b)
                local.append(plsc.sort_key_val(acc, iota + g * 16,
                                               descending=True))
            pair = _merge_desc(*local[0], *local[1])
            run = pair if run is None else _merge_desc(*run, *pair)
        _, top_ids = run
        cand = None
        for j in range(_TOPC):
            cid = top_ids[j]
            g = lax.shift_right_logical(cid, 4)
            l = jnp.bitwise_and(cid, 15)
            pos = g * 256 + l + iota * 16
            v = plsc.load_gather(bi, [jnp.full((_LANES,), rl, jnp.int32),
                                      pos])
            v = jnp.where(pos == r, neg, v)
            sp = plsc.sort_key_val(v, pos, descending=True)
            cand = sp if cand is None else _merge_desc(*cand, *sp)
        _, cols = cand
        colvec = jnp.where(iota < _K, cols, 0)
        pv = plsc.load_gather(bp, [jnp.full((_LANES,), rl, jnp.int32),
                                   colvec])
        vals_v[ci * _CHUNK + rl, pl.ds(0, _LANES)] = pv

    pltpu.async_copy(init_hbm.at[rows_of(0)], bufi.at[0], si0)
    pltpu.async_copy(pred_hbm.at[rows_of(0)], bufp.at[0], sp0)
    pltpu.async_copy(init_hbm.at[rows_of(1)], bufi.at[1], si1)
    pltpu.async_copy(pred_hbm.at[rows_of(1)], bufp.at[1], sp1)

    def half_body(h, carry):
        for slot, si, sp in ((0, si0, sp0), (1, si1, sp1)):
            ci = h * 2 + slot
            pltpu.make_async_copy(init_hbm.at[rows_of(ci)], bufi.at[slot],
                                  si).wait()
            pltpu.make_async_copy(pred_hbm.at[rows_of(ci)], bufp.at[slot],
                                  sp).wait()

            def row_body(rl, c2, _s=slot):
                do_row(bufi.at[_s], bufp.at[_s], rl, c2)
                return c2

            lax.fori_loop(0, _CHUNK, row_body, ci)
            nci = ci + 2

            @pl.when(nci < _NCHUNK)
            def _(slot=slot, si=si, sp=sp, nci=nci):
                pltpu.async_copy(init_hbm.at[rows_of(nci)], bufi.at[slot],
                                 si)
                pltpu.async_copy(pred_hbm.at[rows_of(nci)], bufp.at[slot],
                                 sp)
        return carry

    lax.fori_loop(0, _NCHUNK // 2, half_body, 0)
    pltpu.sync_copy(vals_v, out_hbm.at[pl.ds(row0, _ROWS_PER_W)])


def kernel(affinity_pred, affinity_init):
    denom = pl.pallas_call(
        _denom_kernel,
        grid=(16,),
        in_specs=[pl.BlockSpec((_N // 16, _N), lambda i: (i, 0))],
        out_specs=pl.BlockSpec((_N // 16, 1), lambda i: (i, 0)),
        out_shape=jax.ShapeDtypeStruct((_N, 1), jnp.float32),
    )(affinity_pred)

    mesh = plsc.VectorSubcoreMesh(core_axis_name="c", subcore_axis_name="s")
    vals = pl.kernel(
        _sc_top5_gather,
        out_type=jax.ShapeDtypeStruct((_N, _LANES), jnp.float32),
        mesh=mesh,
        compiler_params=pltpu.CompilerParams(needs_layout_passes=False),
        scratch_types=[
            pltpu.VMEM((2, _CHUNK, _N), jnp.float32),
            pltpu.VMEM((2, _CHUNK, _N), jnp.float32),
            pltpu.VMEM((_ROWS_PER_W, _LANES), jnp.float32),
            pltpu.SemaphoreType.DMA,
            pltpu.SemaphoreType.DMA,
            pltpu.SemaphoreType.DMA,
            pltpu.SemaphoreType.DMA,
        ],
    )(affinity_init, affinity_pred)

    loss = pl.pallas_call(
        _finale_kernel,
        grid=(1,),
        in_specs=[
            pl.BlockSpec((_N, _LANES), lambda i: (0, 0)),
            pl.BlockSpec((_N, 1), lambda i: (0, 0)),
        ],
        out_specs=pl.BlockSpec((1, 1), lambda i: (0, 0)),
        out_shape=jax.ShapeDtypeStruct((1, 1), jnp.float32),
    )(vals, denom)
    return loss[0, 0]
